# baseline (speedup 1.0000x reference)
import functools

import jax
import jax.numpy as jnp
from jax import lax
from jax.experimental import pallas as pl
from jax.experimental.pallas import tpu as pltpu
from jax.experimental.pallas import tpu_sc as plsc

N = 10000
E = 320000
D = 128

_NC, _NS = 2, 16
_NW = _NC * _NS
_EPW = E // _NW
_K = 125
_NCH = _EPW // _K
_PH = 2
_CPP = _NCH // _PH
_NP = 10240
_RPT = _NP // _NS


def _seg_body(y_hbm, src_hbm, dst_hbm, z128_hbm,
              sums_hbm, srcv, dstv, rows0, rows1, accum,
              sem0, sem1, sems0, sems1):
    c = lax.axis_index("c")
    s = lax.axis_index("s")
    w = s * _NC + c

    def gat(g, rows, sem):
        return pltpu.async_copy(y_hbm.at[srcv.at[g]], rows, sem)

    def gat_wait(g, rows, sem):
        pltpu.make_async_copy(y_hbm.at[srcv.at[g]], rows, sem).wait()

    def sca(g, rows, sem):
        return pltpu.async_copy(rows, accum.at[dstv.at[g]], sem, add=True)

    def sca_wait(g, rows, sem):
        pltpu.make_async_copy(rows, accum.at[dstv.at[g]], sem).wait()

    pltpu.sync_copy(z128_hbm, accum.at[pl.ds(s * _RPT, _RPT)])
    plsc.subcore_barrier()

    for p in range(_PH):
        pltpu.sync_copy(src_hbm.at[w, pl.ds(p * _CPP, _CPP)], srcv)
        pltpu.sync_copy(dst_hbm.at[w, pl.ds(p * _CPP, _CPP)], dstv)

        gat(0, rows0, sem0)
        gat_wait(0, rows0, sem0)
        sca(0, rows0, sems0)
        gat(1, rows1, sem1)

        def pair(i, carry):
            g = 2 * i
            gat_wait(g + 1, rows1, sem1)
            sca(g + 1, rows1, sems1)
            sca_wait(g, rows0, sems0)
            gat(g + 2, rows0, sem0)
            gat_wait(g + 2, rows0, sem0)
            sca(g + 2, rows0, sems0)
            sca_wait(g + 1, rows1, sems1)
            gat(g + 3, rows1, sem1)
            return carry

        lax.fori_loop(0, _CPP // 2 - 1, pair, 0)
        g = _CPP - 2
        gat_wait(g + 1, rows1, sem1)
        sca(g + 1, rows1, sems1)
        sca_wait(g, rows0, sems0)
        sca_wait(g + 1, rows1, sems1)

    plsc.subcore_barrier()

    pltpu.sync_copy(accum.at[pl.ds(s * _RPT, _RPT)],
                    sums_hbm.at[c, pl.ds(s * _RPT, _RPT)])


@functools.lru_cache(maxsize=None)
def _make_seg():
    mesh = plsc.VectorSubcoreMesh(core_axis_name="c", subcore_axis_name="s",
                                  num_cores=_NC, num_subcores=_NS)
    return pl.kernel(
        _seg_body,
        out_type=jax.ShapeDtypeStruct((_NC, _NP, D), jnp.float32),
        mesh=mesh,
        scratch_types=[
            pltpu.VMEM((_CPP, _K), jnp.int32),
            pltpu.VMEM((_CPP, _K), jnp.int32),
            pltpu.VMEM((_K, D), jnp.float32),
            pltpu.VMEM((_K, D), jnp.float32),
            pltpu.VMEM_SHARED((_NP, D), jnp.float32),
            pltpu.SemaphoreType.DMA,
            pltpu.SemaphoreType.DMA,
            pltpu.SemaphoreType.DMA,
            pltpu.SemaphoreType.DMA,
        ],
        name="seg_sum_sc",
    )


def _cnt_body(dst_hbm, z128_hbm, ones_hbm, cnt_hbm, dstv, onesv, accum_cnt, sem):
    c = lax.axis_index("c")
    s = lax.axis_index("s")
    w = s * _NC + c

    pltpu.sync_copy(z128_hbm, accum_cnt.at[pl.ds(s * _RPT, _RPT)])
    pltpu.sync_copy(ones_hbm, onesv)
    pltpu.sync_copy(dst_hbm.at[w], dstv)
    plsc.subcore_barrier()

    def chunk(g, carry):
        pltpu.sync_copy(onesv, accum_cnt.at[dstv.at[g]], add=True)
        return carry

    lax.fori_loop(0, _NCH, chunk, 0)
    plsc.subcore_barrier()

    pltpu.sync_copy(accum_cnt.at[pl.ds(s * _RPT, _RPT)],
                    cnt_hbm.at[c, pl.ds(s * _RPT, _RPT)])


@functools.lru_cache(maxsize=None)
def _make_cnt():
    mesh = plsc.VectorSubcoreMesh(core_axis_name="c", subcore_axis_name="s",
                                  num_cores=_NC, num_subcores=_NS)
    return pl.kernel(
        _cnt_body,
        out_type=jax.ShapeDtypeStruct((_NC, _NP, D), jnp.float32),
        mesh=mesh,
        scratch_types=[
            pltpu.VMEM((_NCH, _K), jnp.int32),
            pltpu.VMEM((_K, D), jnp.float32),
            pltpu.VMEM_SHARED((_NP, D), jnp.float32),
            pltpu.SemaphoreType.DMA,
        ],
        name="cnt_sc",
    )


_R = 400
_NT = N // _R


def _k1_body(x_ref, we_ref, be_ref, wl1_ref, h_ref, y1_ref):
    h = jnp.maximum(
        jnp.dot(x_ref[...], we_ref[...], preferred_element_type=jnp.float32)
        + be_ref[...], 0.0)
    h_ref[...] = h
    y1_ref[...] = jnp.dot(h, wl1_ref[...], preferred_element_type=jnp.float32)


def _k2_body(sums_ref, cnt_ref, h_ref, wr1_ref, b1_ref, wl2_ref, h1_ref, y2_ref):
    sums = sums_ref[0] + sums_ref[1]
    sc128 = jnp.sum(cnt_ref[0] + cnt_ref[1], axis=1, keepdims=True)
    inv = 1.0 / jnp.maximum(sc128 * 0.0078125, 1.0)
    h1 = jnp.maximum(
        sums * inv
        + jnp.dot(h_ref[...], wr1_ref[...], preferred_element_type=jnp.float32)
        + b1_ref[...], 0.0)
    h1_ref[...] = h1
    y2_ref[...] = jnp.dot(h1, wl2_ref[...], preferred_element_type=jnp.float32)


def _k3_body(sums_ref, cnt_ref, h1_ref, wr2_ref, b2_ref, wc1_ref, bc1_ref,
             wc2_ref, bc2_ref, out_ref, g_ref):
    i = pl.program_id(0)
    sums = sums_ref[0] + sums_ref[1]
    sc128 = jnp.sum(cnt_ref[0] + cnt_ref[1], axis=1, keepdims=True)
    inv = 1.0 / jnp.maximum(sc128 * 0.0078125, 1.0)
    h2 = jnp.maximum(
        sums * inv
        + jnp.dot(h1_ref[...], wr2_ref[...], preferred_element_type=jnp.float32)
        + b2_ref[...], 0.0)

    @pl.when(i == 0)
    def _():
        g_ref[...] = jnp.zeros_like(g_ref)

    g_ref[...] += jnp.sum(h2, axis=0, keepdims=True)

    @pl.when(i == _NT - 1)
    def _():
        g = g_ref[...]
        t = jnp.maximum(
            jnp.dot(g, wc1_ref[...], preferred_element_type=jnp.float32)
            + bc1_ref[...], 0.0)
        out_ref[...] = (jnp.dot(t, wc2_ref[...], preferred_element_type=jnp.float32)
                        + bc2_ref[...])


def _full(shape):
    return pl.BlockSpec(shape, lambda i: (0,) * len(shape))


_k1 = pl.pallas_call(
    _k1_body,
    grid=(_NT,),
    in_specs=[
        pl.BlockSpec((_R, D), lambda i: (i, 0)),
        _full((D, D)), _full((1, D)), _full((D, D)),
    ],
    out_specs=[
        pl.BlockSpec((_R, D), lambda i: (i, 0)),
        pl.BlockSpec((_R, D), lambda i: (i, 0)),
    ],
    out_shape=[
        jax.ShapeDtypeStruct((N, D), jnp.float32),
        jax.ShapeDtypeStruct((N, D), jnp.float32),
    ],
)

_k2 = pl.pallas_call(
    _k2_body,
    grid=(_NT,),
    in_specs=[
        pl.BlockSpec((_NC, _R, D), lambda i: (0, i, 0)),
        pl.BlockSpec((_NC, _R, D), lambda i: (0, i, 0)),
        pl.BlockSpec((_R, D), lambda i: (i, 0)),
        _full((D, D)), _full((1, D)), _full((D, D)),
    ],
    out_specs=[
        pl.BlockSpec((_R, D), lambda i: (i, 0)),
        pl.BlockSpec((_R, D), lambda i: (i, 0)),
    ],
    out_shape=[
        jax.ShapeDtypeStruct((N, D), jnp.float32),
        jax.ShapeDtypeStruct((N, D), jnp.float32),
    ],
)

_k3 = pl.pallas_call(
    _k3_body,
    grid=(_NT,),
    in_specs=[
        pl.BlockSpec((_NC, _R, D), lambda i: (0, i, 0)),
        pl.BlockSpec((_NC, _R, D), lambda i: (0, i, 0)),
        pl.BlockSpec((_R, D), lambda i: (i, 0)),
        _full((D, D)), _full((1, D)),
        _full((D, 64)), _full((1, 64)),
        _full((64, 128)), _full((1, 128)),
    ],
    out_specs=pl.BlockSpec((1, 128), lambda i: (0, 0)),
    out_shape=jax.ShapeDtypeStruct((1, 128), jnp.float32),
    scratch_shapes=[pltpu.VMEM((1, D), jnp.float32)],
)

def kernel(x, edge_index, W_enc, b_enc, Wl1, Wr1, b1, Wl2, Wr2, b2,
           Wlp, Wrp, bp, Wc1, bc1, Wc2, bc2):
    src = edge_index[0].reshape(_NW, _NCH, _K)
    dst = edge_index[1].reshape(_NW, _NCH, _K)
    z128 = jnp.zeros((_RPT, D), jnp.float32)
    ones = jnp.ones((_K, D), jnp.float32)

    h, y1 = _k1(x, W_enc, b_enc.reshape(1, D), Wl1)
    cnt = _make_cnt()(dst, z128, ones)
    sums1 = _make_seg()(y1, src, dst, z128)
    h1, y2 = _k2(sums1, cnt, h, Wr1, b1.reshape(1, D), Wl2)
    sums2 = _make_seg()(y2, src, dst, z128)
    wc2p = jnp.zeros((64, 128), jnp.float32).at[:, :10].set(Wc2)
    bc2p = jnp.zeros((1, 128), jnp.float32).at[0, :10].set(bc2)
    out = _k3(sums2, cnt, h1, Wr2, b2.reshape(1, D), Wc1, bc1.reshape(1, 64),
              wc2p, bc2p)
    return out[0, :10]


# speedup vs baseline: 1.1105x; 1.1105x over previous
import functools

import jax
import jax.numpy as jnp
from jax import lax
from jax.experimental import pallas as pl
from jax.experimental.pallas import tpu as pltpu
from jax.experimental.pallas import tpu_sc as plsc

N = 10000
E = 320000
D = 128

_NC, _NS = 2, 16
_NW = _NC * _NS
_EPW = E // _NW
_K = 125
_NCH = _EPW // _K
_PH = 2
_CPP = _NCH // _PH
_NP = 10240
_RPT = _NP // _NS


def _seg_body(y_hbm, src_hbm, dst_hbm, z128_hbm,
              sums_hbm, srcv, dstv, rows0, rows1, accum, sem0, sem1):
    c = lax.axis_index("c")
    s = lax.axis_index("s")
    w = s * _NC + c

    pltpu.sync_copy(z128_hbm, accum.at[pl.ds(s * _RPT, _RPT)])
    plsc.subcore_barrier()

    for p in range(_PH):
        pltpu.sync_copy(src_hbm.at[w, pl.ds(p * _CPP, _CPP)], srcv)
        pltpu.sync_copy(dst_hbm.at[w, pl.ds(p * _CPP, _CPP)], dstv)

        pltpu.async_copy(y_hbm.at[srcv.at[0]], rows0, sem0)

        def pair(i, carry):
            g = 2 * i
            pltpu.async_copy(y_hbm.at[srcv.at[g + 1]], rows1, sem1)
            pltpu.make_async_copy(y_hbm.at[srcv.at[g]], rows0, sem0).wait()
            pltpu.sync_copy(rows0, accum.at[dstv.at[g]], add=True)
            pltpu.async_copy(y_hbm.at[srcv.at[g + 2]], rows0, sem0)
            pltpu.make_async_copy(y_hbm.at[srcv.at[g + 1]], rows1, sem1).wait()
            pltpu.sync_copy(rows1, accum.at[dstv.at[g + 1]], add=True)
            return carry

        lax.fori_loop(0, _CPP // 2 - 1, pair, 0)
        g = _CPP - 2
        pltpu.async_copy(y_hbm.at[srcv.at[g + 1]], rows1, sem1)
        pltpu.make_async_copy(y_hbm.at[srcv.at[g]], rows0, sem0).wait()
        pltpu.sync_copy(rows0, accum.at[dstv.at[g]], add=True)
        pltpu.make_async_copy(y_hbm.at[srcv.at[g + 1]], rows1, sem1).wait()
        pltpu.sync_copy(rows1, accum.at[dstv.at[g + 1]], add=True)

    plsc.subcore_barrier()

    pltpu.sync_copy(accum.at[pl.ds(s * _RPT, _RPT)],
                    sums_hbm.at[c, pl.ds(s * _RPT, _RPT)])


@functools.lru_cache(maxsize=None)
def _make_seg():
    mesh = plsc.VectorSubcoreMesh(core_axis_name="c", subcore_axis_name="s",
                                  num_cores=_NC, num_subcores=_NS)
    return pl.kernel(
        _seg_body,
        out_type=jax.ShapeDtypeStruct((_NC, _NP, D), jnp.float32),
        mesh=mesh,
        scratch_types=[
            pltpu.VMEM((_CPP, _K), jnp.int32),
            pltpu.VMEM((_CPP, _K), jnp.int32),
            pltpu.VMEM((_K, D), jnp.float32),
            pltpu.VMEM((_K, D), jnp.float32),
            pltpu.VMEM_SHARED((_NP, D), jnp.float32),
            pltpu.SemaphoreType.DMA,
            pltpu.SemaphoreType.DMA,
        ],
        name="seg_sum_sc",
    )


def _cnt_body(dst_hbm, z128_hbm, ones_hbm, cnt_hbm, dstv, onesv, accum_cnt, sem):
    c = lax.axis_index("c")
    s = lax.axis_index("s")
    w = s * _NC + c

    pltpu.sync_copy(z128_hbm, accum_cnt.at[pl.ds(s * _RPT, _RPT)])
    pltpu.sync_copy(ones_hbm, onesv)
    pltpu.sync_copy(dst_hbm.at[w], dstv)
    plsc.subcore_barrier()

    def grp(j, carry):
        base = j * 8
        for t in range(8):
            pltpu.async_copy(onesv, accum_cnt.at[dstv.at[base + t]], sem,
                             add=True)
        for t in range(8):
            pltpu.make_async_copy(onesv, accum_cnt.at[dstv.at[base + t]],
                                  sem).wait()
        return carry

    lax.fori_loop(0, _NCH // 8, grp, 0)
    plsc.subcore_barrier()

    pltpu.sync_copy(accum_cnt.at[pl.ds(s * _RPT, _RPT)],
                    cnt_hbm.at[c, pl.ds(s * _RPT, _RPT)])


@functools.lru_cache(maxsize=None)
def _make_cnt():
    mesh = plsc.VectorSubcoreMesh(core_axis_name="c", subcore_axis_name="s",
                                  num_cores=_NC, num_subcores=_NS)
    return pl.kernel(
        _cnt_body,
        out_type=jax.ShapeDtypeStruct((_NC, _NP, D), jnp.float32),
        mesh=mesh,
        scratch_types=[
            pltpu.VMEM((_NCH, _K), jnp.int32),
            pltpu.VMEM((_K, D), jnp.float32),
            pltpu.VMEM_SHARED((_NP, D), jnp.float32),
            pltpu.SemaphoreType.DMA,
        ],
        name="cnt_sc",
    )


_R = 400
_NT = N // _R


def _k1_body(x_ref, we_ref, be_ref, wl1_ref, h_ref, y1_ref):
    h = jnp.maximum(
        jnp.dot(x_ref[...], we_ref[...], preferred_element_type=jnp.float32)
        + be_ref[...], 0.0)
    h_ref[...] = h
    y1_ref[...] = jnp.dot(h, wl1_ref[...], preferred_element_type=jnp.float32)


def _k2_body(sums_ref, cnt_ref, h_ref, wr1_ref, b1_ref, wl2_ref, h1_ref, y2_ref):
    sums = sums_ref[0] + sums_ref[1]
    sc128 = jnp.sum(cnt_ref[0] + cnt_ref[1], axis=1, keepdims=True)
    inv = 1.0 / jnp.maximum(sc128 * 0.0078125, 1.0)
    h1 = jnp.maximum(
        sums * inv
        + jnp.dot(h_ref[...], wr1_ref[...], preferred_element_type=jnp.float32)
        + b1_ref[...], 0.0)
    h1_ref[...] = h1
    y2_ref[...] = jnp.dot(h1, wl2_ref[...], preferred_element_type=jnp.float32)


def _k3_body(sums_ref, cnt_ref, h1_ref, wr2_ref, b2_ref, wc1_ref, bc1_ref,
             wc2_ref, bc2_ref, out_ref, g_ref):
    i = pl.program_id(0)
    sums = sums_ref[0] + sums_ref[1]
    sc128 = jnp.sum(cnt_ref[0] + cnt_ref[1], axis=1, keepdims=True)
    inv = 1.0 / jnp.maximum(sc128 * 0.0078125, 1.0)
    h2 = jnp.maximum(
        sums * inv
        + jnp.dot(h1_ref[...], wr2_ref[...], preferred_element_type=jnp.float32)
        + b2_ref[...], 0.0)

    @pl.when(i == 0)
    def _():
        g_ref[...] = jnp.zeros_like(g_ref)

    g_ref[...] += jnp.sum(h2, axis=0, keepdims=True)

    @pl.when(i == _NT - 1)
    def _():
        g = g_ref[...]
        t = jnp.maximum(
            jnp.dot(g, wc1_ref[...], preferred_element_type=jnp.float32)
            + bc1_ref[...], 0.0)
        out_ref[...] = (jnp.dot(t, wc2_ref[...], preferred_element_type=jnp.float32)
                        + bc2_ref[...])


def _full(shape):
    return pl.BlockSpec(shape, lambda i: (0,) * len(shape))


_k1 = pl.pallas_call(
    _k1_body,
    grid=(_NT,),
    in_specs=[
        pl.BlockSpec((_R, D), lambda i: (i, 0)),
        _full((D, D)), _full((1, D)), _full((D, D)),
    ],
    out_specs=[
        pl.BlockSpec((_R, D), lambda i: (i, 0)),
        pl.BlockSpec((_R, D), lambda i: (i, 0)),
    ],
    out_shape=[
        jax.ShapeDtypeStruct((N, D), jnp.float32),
        jax.ShapeDtypeStruct((N, D), jnp.float32),
    ],
)

_k2 = pl.pallas_call(
    _k2_body,
    grid=(_NT,),
    in_specs=[
        pl.BlockSpec((_NC, _R, D), lambda i: (0, i, 0)),
        pl.BlockSpec((_NC, _R, D), lambda i: (0, i, 0)),
        pl.BlockSpec((_R, D), lambda i: (i, 0)),
        _full((D, D)), _full((1, D)), _full((D, D)),
    ],
    out_specs=[
        pl.BlockSpec((_R, D), lambda i: (i, 0)),
        pl.BlockSpec((_R, D), lambda i: (i, 0)),
    ],
    out_shape=[
        jax.ShapeDtypeStruct((N, D), jnp.float32),
        jax.ShapeDtypeStruct((N, D), jnp.float32),
    ],
)

_k3 = pl.pallas_call(
    _k3_body,
    grid=(_NT,),
    in_specs=[
        pl.BlockSpec((_NC, _R, D), lambda i: (0, i, 0)),
        pl.BlockSpec((_NC, _R, D), lambda i: (0, i, 0)),
        pl.BlockSpec((_R, D), lambda i: (i, 0)),
        _full((D, D)), _full((1, D)),
        _full((D, 64)), _full((1, 64)),
        _full((64, 128)), _full((1, 128)),
    ],
    out_specs=pl.BlockSpec((1, 128), lambda i: (0, 0)),
    out_shape=jax.ShapeDtypeStruct((1, 128), jnp.float32),
    scratch_shapes=[pltpu.VMEM((1, D), jnp.float32)],
)

def kernel(x, edge_index, W_enc, b_enc, Wl1, Wr1, b1, Wl2, Wr2, b2,
           Wlp, Wrp, bp, Wc1, bc1, Wc2, bc2):
    src = edge_index[0].reshape(_NW, _NCH, _K)
    dst = edge_index[1].reshape(_NW, _NCH, _K)
    z128 = jnp.zeros((_RPT, D), jnp.float32)
    ones = jnp.ones((_K, D), jnp.float32)

    cnt = _make_cnt()(dst, z128, ones)
    h, y1 = _k1(x, W_enc, b_enc.reshape(1, D), Wl1)
    sums1 = _make_seg()(y1, src, dst, z128)
    h1, y2 = _k2(sums1, cnt, h, Wr1, b1.reshape(1, D), Wl2)
    sums2 = _make_seg()(y2, src, dst, z128)
    wc2p = jnp.zeros((64, 128), jnp.float32).at[:, :10].set(Wc2)
    bc2p = jnp.zeros((1, 128), jnp.float32).at[0, :10].set(bc2)
    out = _k3(sums2, cnt, h1, Wr2, b2.reshape(1, D), Wc1, bc1.reshape(1, 64),
              wc2p, bc2p)
    return out[0, :10]


# speedup vs baseline: 1.1118x; 1.0012x over previous
import functools

import jax
import jax.numpy as jnp
from jax import lax
from jax.experimental import pallas as pl
from jax.experimental.pallas import tpu as pltpu
from jax.experimental.pallas import tpu_sc as plsc

N = 10000
E = 320000
D = 128

_NC, _NS = 2, 16
_NW = _NC * _NS
_EPW = E // _NW
_K = 125
_NCH = _EPW // _K
_PH = 2
_CPP = _NCH // _PH
_NP = 10240
_RPT = _NP // _NS


def _seg_body(with_cnt, y_hbm, src_hbm, dst_hbm, z128_hbm, ones_hbm,
              *refs):
    if with_cnt:
        sums_hbm, cnt_hbm, srcv, dstv, rows0, rows1, accum, sem0, sem1 = refs
    else:
        sums_hbm, srcv, dstv, rows0, rows1, accum, sem0, sem1 = refs
    c = lax.axis_index("c")
    s = lax.axis_index("s")
    w = s * _NC + c

    if with_cnt:
        pltpu.sync_copy(z128_hbm, accum.at[pl.ds(s * _RPT, _RPT)])
        pltpu.sync_copy(ones_hbm, rows0)
        plsc.subcore_barrier()
        for p in range(_PH):
            pltpu.sync_copy(dst_hbm.at[w, pl.ds(p * _CPP, _CPP)], dstv)

            def grp(j, carry):
                base = j * 8
                for t in range(8):
                    pltpu.async_copy(rows0, accum.at[dstv.at[base + t]],
                                     sem0, add=True)
                for t in range(8):
                    pltpu.make_async_copy(rows0, accum.at[dstv.at[base + t]],
                                          sem0).wait()
                return carry

            lax.fori_loop(0, _CPP // 8, grp, 0)
        plsc.subcore_barrier()
        pltpu.sync_copy(accum.at[pl.ds(s * _RPT, _RPT)],
                        cnt_hbm.at[c, pl.ds(s * _RPT, _RPT)])
        plsc.subcore_barrier()

    pltpu.sync_copy(z128_hbm, accum.at[pl.ds(s * _RPT, _RPT)])
    plsc.subcore_barrier()

    for p in range(_PH):
        pltpu.sync_copy(src_hbm.at[w, pl.ds(p * _CPP, _CPP)], srcv)
        pltpu.sync_copy(dst_hbm.at[w, pl.ds(p * _CPP, _CPP)], dstv)

        pltpu.async_copy(y_hbm.at[srcv.at[0]], rows0, sem0)

        def pair(i, carry):
            g = 2 * i
            pltpu.async_copy(y_hbm.at[srcv.at[g + 1]], rows1, sem1)
            pltpu.make_async_copy(y_hbm.at[srcv.at[g]], rows0, sem0).wait()
            pltpu.sync_copy(rows0, accum.at[dstv.at[g]], add=True)
            pltpu.async_copy(y_hbm.at[srcv.at[g + 2]], rows0, sem0)
            pltpu.make_async_copy(y_hbm.at[srcv.at[g + 1]], rows1, sem1).wait()
            pltpu.sync_copy(rows1, accum.at[dstv.at[g + 1]], add=True)
            return carry

        lax.fori_loop(0, _CPP // 2 - 1, pair, 0)
        g = _CPP - 2
        pltpu.async_copy(y_hbm.at[srcv.at[g + 1]], rows1, sem1)
        pltpu.make_async_copy(y_hbm.at[srcv.at[g]], rows0, sem0).wait()
        pltpu.sync_copy(rows0, accum.at[dstv.at[g]], add=True)
        pltpu.make_async_copy(y_hbm.at[srcv.at[g + 1]], rows1, sem1).wait()
        pltpu.sync_copy(rows1, accum.at[dstv.at[g + 1]], add=True)

    plsc.subcore_barrier()

    pltpu.sync_copy(accum.at[pl.ds(s * _RPT, _RPT)],
                    sums_hbm.at[c, pl.ds(s * _RPT, _RPT)])


@functools.lru_cache(maxsize=None)
def _make_seg(with_cnt):
    mesh = plsc.VectorSubcoreMesh(core_axis_name="c", subcore_axis_name="s",
                                  num_cores=_NC, num_subcores=_NS)
    out = jax.ShapeDtypeStruct((_NC, _NP, D), jnp.float32)
    return pl.kernel(
        functools.partial(_seg_body, with_cnt),
        out_type=(out, out) if with_cnt else out,
        mesh=mesh,
        scratch_types=[
            pltpu.VMEM((_CPP, _K), jnp.int32),
            pltpu.VMEM((_CPP, _K), jnp.int32),
            pltpu.VMEM((_K, D), jnp.float32),
            pltpu.VMEM((_K, D), jnp.float32),
            pltpu.VMEM_SHARED((_NP, D), jnp.float32),
            pltpu.SemaphoreType.DMA,
            pltpu.SemaphoreType.DMA,
        ],
        name="seg_cnt_sc" if with_cnt else "seg_sum_sc",
    )


_R = 400
_NT = N // _R


def _k1_body(x_ref, we_ref, be_ref, wl1_ref, h_ref, y1_ref):
    h = jnp.maximum(
        jnp.dot(x_ref[...], we_ref[...], preferred_element_type=jnp.float32)
        + be_ref[...], 0.0)
    h_ref[...] = h
    y1_ref[...] = jnp.dot(h, wl1_ref[...], preferred_element_type=jnp.float32)


def _k2_body(sums_ref, cnt_ref, h_ref, wr1_ref, b1_ref, wl2_ref, h1_ref, y2_ref):
    sums = sums_ref[0] + sums_ref[1]
    sc128 = jnp.sum(cnt_ref[0] + cnt_ref[1], axis=1, keepdims=True)
    inv = 1.0 / jnp.maximum(sc128 * 0.0078125, 1.0)
    h1 = jnp.maximum(
        sums * inv
        + jnp.dot(h_ref[...], wr1_ref[...], preferred_element_type=jnp.float32)
        + b1_ref[...], 0.0)
    h1_ref[...] = h1
    y2_ref[...] = jnp.dot(h1, wl2_ref[...], preferred_element_type=jnp.float32)


def _k3_body(sums_ref, cnt_ref, h1_ref, wr2_ref, b2_ref, wc1_ref, bc1_ref,
             wc2_ref, bc2_ref, out_ref, g_ref):
    i = pl.program_id(0)
    sums = sums_ref[0] + sums_ref[1]
    sc128 = jnp.sum(cnt_ref[0] + cnt_ref[1], axis=1, keepdims=True)
    inv = 1.0 / jnp.maximum(sc128 * 0.0078125, 1.0)
    h2 = jnp.maximum(
        sums * inv
        + jnp.dot(h1_ref[...], wr2_ref[...], preferred_element_type=jnp.float32)
        + b2_ref[...], 0.0)

    @pl.when(i == 0)
    def _():
        g_ref[...] = jnp.zeros_like(g_ref)

    g_ref[...] += jnp.sum(h2, axis=0, keepdims=True)

    @pl.when(i == _NT - 1)
    def _():
        g = g_ref[...]
        t = jnp.maximum(
            jnp.dot(g, wc1_ref[...], preferred_element_type=jnp.float32)
            + bc1_ref[...], 0.0)
        out_ref[...] = (jnp.dot(t, wc2_ref[...], preferred_element_type=jnp.float32)
                        + bc2_ref[...])


def _full(shape):
    return pl.BlockSpec(shape, lambda i: (0,) * len(shape))


_k1 = pl.pallas_call(
    _k1_body,
    grid=(_NT,),
    in_specs=[
        pl.BlockSpec((_R, D), lambda i: (i, 0)),
        _full((D, D)), _full((1, D)), _full((D, D)),
    ],
    out_specs=[
        pl.BlockSpec((_R, D), lambda i: (i, 0)),
        pl.BlockSpec((_R, D), lambda i: (i, 0)),
    ],
    out_shape=[
        jax.ShapeDtypeStruct((N, D), jnp.float32),
        jax.ShapeDtypeStruct((N, D), jnp.float32),
    ],
)

_k2 = pl.pallas_call(
    _k2_body,
    grid=(_NT,),
    in_specs=[
        pl.BlockSpec((_NC, _R, D), lambda i: (0, i, 0)),
        pl.BlockSpec((_NC, _R, D), lambda i: (0, i, 0)),
        pl.BlockSpec((_R, D), lambda i: (i, 0)),
        _full((D, D)), _full((1, D)), _full((D, D)),
    ],
    out_specs=[
        pl.BlockSpec((_R, D), lambda i: (i, 0)),
        pl.BlockSpec((_R, D), lambda i: (i, 0)),
    ],
    out_shape=[
        jax.ShapeDtypeStruct((N, D), jnp.float32),
        jax.ShapeDtypeStruct((N, D), jnp.float32),
    ],
)

_k3 = pl.pallas_call(
    _k3_body,
    grid=(_NT,),
    in_specs=[
        pl.BlockSpec((_NC, _R, D), lambda i: (0, i, 0)),
        pl.BlockSpec((_NC, _R, D), lambda i: (0, i, 0)),
        pl.BlockSpec((_R, D), lambda i: (i, 0)),
        _full((D, D)), _full((1, D)),
        _full((D, 64)), _full((1, 64)),
        _full((64, 128)), _full((1, 128)),
    ],
    out_specs=pl.BlockSpec((1, 128), lambda i: (0, 0)),
    out_shape=jax.ShapeDtypeStruct((1, 128), jnp.float32),
    scratch_shapes=[pltpu.VMEM((1, D), jnp.float32)],
)

def kernel(x, edge_index, W_enc, b_enc, Wl1, Wr1, b1, Wl2, Wr2, b2,
           Wlp, Wrp, bp, Wc1, bc1, Wc2, bc2):
    src = edge_index[0].reshape(_NW, _NCH, _K)
    dst = edge_index[1].reshape(_NW, _NCH, _K)
    z128 = jnp.zeros((_RPT, D), jnp.float32)
    ones = jnp.ones((_K, D), jnp.float32)

    h, y1 = _k1(x, W_enc, b_enc.reshape(1, D), Wl1)
    sums1, cnt = _make_seg(True)(y1, src, dst, z128, ones)
    h1, y2 = _k2(sums1, cnt, h, Wr1, b1.reshape(1, D), Wl2)
    sums2 = _make_seg(False)(y2, src, dst, z128, ones)
    wc2p = jnp.zeros((64, 128), jnp.float32).at[:, :10].set(Wc2)
    bc2p = jnp.zeros((1, 128), jnp.float32).at[0, :10].set(bc2)
    out = _k3(sums2, cnt, h1, Wr2, b2.reshape(1, D), Wc1, bc1.reshape(1, 64),
              wc2p, bc2p)
    return out[0, :10]
